# baseline (speedup 1.0000x reference)
import functools

import jax
import jax.numpy as jnp
from jax import lax
from jax.experimental import pallas as pl
from jax.experimental.pallas import tpu as pltpu
from jax.experimental.pallas import tpu_sc as plsc

_VOCAB = 1000000
_DIM = 64
_BATCH = 4096
_WINDOW = 20
_NEGS = 5
_BW = _BATCH * _WINDOW

_info = plsc.get_sparse_core_info()
_NC, _NS = _info.num_cores, _info.num_subcores
_NW = _NC * _NS
_CHUNK = _BATCH // _NW
_KPT = _BW // _NW
_OROWS = 16
_OSUB = _OROWS * _WINDOW
_NSUB = _KPT // _OSUB

def _sc_scores(iword2, owords, nwords, W):
    mesh = plsc.VectorSubcoreMesh(core_axis_name="c", subcore_axis_name="s")

    @functools.partial(
        pl.kernel,
        mesh=mesh,
        out_type=jax.ShapeDtypeStruct((6 * _BW,), jnp.float32),
        scratch_types=[
            pltpu.VMEM((_CHUNK,), jnp.int32),
            pltpu.VMEM((_CHUNK, _DIM), jnp.float32),
            [pltpu.VMEM((_CHUNK, _NEGS), jnp.int32)] * 2,
            [pltpu.VMEM((_NEGS * _CHUNK,), jnp.int32)] * 2,
            [pltpu.VMEM((_NEGS * _CHUNK, _DIM), jnp.float32)] * 2,
            [pltpu.VMEM((_OROWS, _WINDOW), jnp.int32)] * 2,
            [pltpu.VMEM((_OSUB,), jnp.int32)] * 2,
            [pltpu.VMEM((_OSUB,), jnp.int32)] * 2,
            pltpu.VMEM((_NEGS, _CHUNK), jnp.float32),
            pltpu.VMEM((_OSUB,), jnp.float32),
            [pltpu.SemaphoreType.DMA] * 2,
        ],
        compiler_params=pltpu.CompilerParams(
            needs_layout_passes=False, use_tc_tiling_on_sc=False),
    )
    def k(iword2_h, owords_h, nwords_h, w_h, out_h,
          iw_idx, iv_rows, n2_idx, n_idx, rows, o2_idx, o_idx, oi_idx,
          sc_n, sc_o, sem):
        wid = lax.axis_index("s") * _NC + lax.axis_index("c")
        iota = lax.iota(jnp.int32, 16)

        base = wid * _CHUNK
        pltpu.sync_copy(iword2_h.at[pl.ds(base, _CHUNK)], iw_idx)
        pltpu.async_copy(w_h.at[iw_idx], iv_rows, sem[0]).wait()

        def n_issue(j, p):
            koff = j * _BATCH + base
            pltpu.sync_copy(nwords_h.at[pl.ds(koff, _CHUNK), :], n2_idx[p])
            for n in range(_NEGS):
                for g in range(_CHUNK // 16):
                    v = plsc.load_gather(
                        n2_idx[p],
                        [g * 16 + iota, jnp.full((16,), n, jnp.int32)])
                    n_idx[p][pl.ds(n * _CHUNK + g * 16, 16)] = v
            for n in range(_NEGS):
                pltpu.async_copy(
                    w_h.at[n_idx[p].at[pl.ds(n * _CHUNK, _CHUNK)]],
                    rows[p].at[pl.ds(n * _CHUNK, _CHUNK)], sem[p])

        def n_wait(p):
            for n in range(_NEGS):
                pltpu.make_async_copy(
                    w_h.at[n_idx[p].at[pl.ds(n * _CHUNK, _CHUNK)]],
                    rows[p].at[pl.ds(n * _CHUNK, _CHUNK)], sem[p]).wait()

        def n_compute(j, p):
            koff = j * _BATCH + base
            ng = rows[p]

            def gbody(g, c2):
                base16 = g * 16
                accs = [jnp.zeros((16,), jnp.float32) for _ in range(_NEGS)]
                for k16 in range(16):
                    kk = base16 + k16
                    lmask = iota == k16
                    iv0 = iv_rows[kk, pl.ds(0, 16)]
                    iv1 = iv_rows[kk, pl.ds(16, 16)]
                    iv2 = iv_rows[kk, pl.ds(32, 16)]
                    iv3 = iv_rows[kk, pl.ds(48, 16)]
                    for n in range(_NEGS):
                        rr = n * _CHUNK + kk
                        v = ng[rr, pl.ds(0, 16)] * iv0
                        v = v + ng[rr, pl.ds(16, 16)] * iv1
                        v = v + ng[rr, pl.ds(32, 16)] * iv2
                        v = v + ng[rr, pl.ds(48, 16)] * iv3
                        accs[n] = jnp.where(lmask, -jnp.sum(v), accs[n])
                for n in range(_NEGS):
                    sc_n[n, pl.ds(base16, 16)] = accs[n]
                return c2

            lax.fori_loop(0, _CHUNK // 16, gbody, 0)
            for n in range(_NEGS):
                pltpu.sync_copy(
                    sc_n.at[n],
                    out_h.at[pl.ds((1 + n) * _BW + koff, _CHUNK)])

        n_issue(0, 0)

        def n_jbody(jj, carry):
            j0 = jj * 2
            n_issue(j0 + 1, 1)
            n_wait(0)
            n_compute(j0, 0)

            @pl.when(jj + 1 < _WINDOW // 2)
            def _():
                n_issue(j0 + 2, 0)

            n_wait(1)
            n_compute(j0 + 1, 1)
            return carry

        lax.fori_loop(0, _WINDOW // 2, n_jbody, 0)

        kpt0 = wid * _KPT

        def o_issue(s, p):
            q0 = wid * (_BATCH // _NW) + s * _OROWS
            pltpu.sync_copy(owords_h.at[pl.ds(q0, _OROWS), :], o2_idx[p])
            def obody(g, c2):
                pos = g * 16 + iota
                row = (pos * 6554) >> 17
                col = pos - row * _WINDOW
                v = plsc.load_gather(o2_idx[p], [row, col])
                o_idx[p][pl.ds(g * 16, 16)] = v
                return c2

            lax.fori_loop(0, _OSUB // 16, obody, 0)
            p0 = lax.rem(kpt0 + s * _OSUB, _BATCH)
            pltpu.sync_copy(iword2_h.at[pl.ds(p0, _OSUB)], oi_idx[p])
            pltpu.async_copy(w_h.at[oi_idx[p]],
                             rows[p].at[pl.ds(0, _OSUB)], sem[p])
            pltpu.async_copy(w_h.at[o_idx[p]],
                             rows[p].at[pl.ds(_OSUB, _OSUB)], sem[p])

        def o_wait(p):
            pltpu.make_async_copy(w_h.at[oi_idx[p]],
                                  rows[p].at[pl.ds(0, _OSUB)], sem[p]).wait()
            pltpu.make_async_copy(w_h.at[o_idx[p]],
                                  rows[p].at[pl.ds(_OSUB, _OSUB)],
                                  sem[p]).wait()

        def o_compute(s, p):
            rw = rows[p]

            def gbody(g, c2):
                base16 = g * 16
                acc = jnp.zeros((16,), jnp.float32)
                for k16 in range(16):
                    kk = base16 + k16
                    v = rw[kk, pl.ds(0, 16)] * rw[_OSUB + kk, pl.ds(0, 16)]
                    v = v + (rw[kk, pl.ds(16, 16)]
                             * rw[_OSUB + kk, pl.ds(16, 16)])
                    v = v + (rw[kk, pl.ds(32, 16)]
                             * rw[_OSUB + kk, pl.ds(32, 16)])
                    v = v + (rw[kk, pl.ds(48, 16)]
                             * rw[_OSUB + kk, pl.ds(48, 16)])
                    acc = jnp.where(iota == k16, jnp.sum(v), acc)
                sc_o[pl.ds(base16, 16)] = acc
                return c2

            lax.fori_loop(0, _OSUB // 16, gbody, 0)
            pltpu.sync_copy(sc_o, out_h.at[pl.ds(kpt0 + s * _OSUB, _OSUB)])

        o_issue(0, 0)

        def o_sbody(ss, carry):
            s0 = ss * 2
            o_issue(s0 + 1, 1)
            o_wait(0)
            o_compute(s0, 0)

            @pl.when(ss + 1 < _NSUB // 2)
            def _():
                o_issue(s0 + 2, 0)

            o_wait(1)
            o_compute(s0 + 1, 1)
            return carry

        lax.fori_loop(0, _NSUB // 2, o_sbody, 0)

    return k(iword2, owords, nwords, W)


_TC_ROWS = 384
_TC_COLS = (6 * _BW) // _TC_ROWS


def _tc_loss(scores):
    def body(s_ref, o_ref):
        x = s_ref[...]
        o_ref[...] = jnp.reshape(
            -jnp.sum(jnp.log(jax.nn.sigmoid(x))) / _BATCH, (1, 1))

    return pl.pallas_call(
        body,
        out_shape=jax.ShapeDtypeStruct((1, 1), jnp.float32),
    )(scores.reshape(_TC_ROWS, _TC_COLS))


def kernel(iword, owords, nwords, W):
    iword2 = jnp.concatenate([iword, iword])
    scores = _sc_scores(iword2, owords, nwords, W)
    return _tc_loss(scores)[0, 0]
